# baseline (speedup 1.0000x reference)
import jax
import jax.numpy as jnp
from jax import lax
from jax.experimental import pallas as pl
from jax.experimental.pallas import tpu as pltpu
from jax.experimental.pallas import tpu_sc as plsc

D = 64
NC = 2
NS = 16
NW = NC * NS
CHUNK = 128
NBUF = 2


def _emb_body(idx_hbm, tbl_hbm, out_hbm, idx_v, buf0, buf1,
              gsem0, gsem1, osem0, osem1):
    nchunk = idx_hbm.shape[1]
    wid = lax.axis_index("s") * NC + lax.axis_index("c")
    pltpu.sync_copy(idx_hbm.at[wid], idx_v)

    bufs = (buf0, buf1)
    gsems = (gsem0, gsem1)
    osems = (osem0, osem1)

    def start_gather(cj, b):
        pltpu.async_copy(tbl_hbm.at[idx_v.at[cj]], bufs[b], gsems[b])

    def wait_gather(b):
        pltpu.make_async_copy(tbl_hbm.at[idx_v.at[0]], bufs[b], gsems[b]).wait()

    def start_store(cj, b):
        pltpu.async_copy(bufs[b], out_hbm.at[wid, cj], osems[b])

    def wait_store(b):
        pltpu.make_async_copy(bufs[b], out_hbm.at[wid, 0], osems[b]).wait()

    for b in range(NBUF):
        start_gather(b, b)

    @pl.loop(0, nchunk - NBUF, step=NBUF)
    def _(j):
        for b in range(NBUF):
            wait_gather(b)
            start_store(j + b, b)
        for b in range(NBUF):
            wait_store(b)
            start_gather(j + NBUF + b, b)

    for b in range(NBUF):
        wait_gather(b)
        start_store(nchunk - NBUF + b, b)
    for b in range(NBUF):
        wait_store(b)


def kernel(input_ids, weight):
    bsz, seq = input_ids.shape
    total = bsz * seq
    assert total % (NW * CHUNK) == 0
    nchunk = total // (NW * CHUNK)
    idx = input_ids.reshape(NW, nchunk, CHUNK).astype(jnp.int32)

    mesh = plsc.VectorSubcoreMesh(core_axis_name="c", subcore_axis_name="s")
    f = pl.kernel(
        _emb_body,
        out_type=jax.ShapeDtypeStruct((NW, nchunk, CHUNK, D), jnp.float32),
        mesh=mesh,
        scratch_types=[
            pltpu.VMEM((nchunk, CHUNK), jnp.int32),
            pltpu.VMEM((CHUNK, D), jnp.float32),
            pltpu.VMEM((CHUNK, D), jnp.float32),
            pltpu.SemaphoreType.DMA,
            pltpu.SemaphoreType.DMA,
            pltpu.SemaphoreType.DMA,
            pltpu.SemaphoreType.DMA,
        ],
        compiler_params=pltpu.CompilerParams(use_tc_tiling_on_sc=False),
    )
    out = f(idx, weight)
    return out.reshape(bsz, seq, D)


# speedup vs baseline: 1.0091x; 1.0091x over previous
import jax
import jax.numpy as jnp
from jax import lax
from jax.experimental import pallas as pl
from jax.experimental.pallas import tpu as pltpu
from jax.experimental.pallas import tpu_sc as plsc

D = 64
NC = 2
NS = 16
NW = NC * NS
CHUNK = 640
NBUF = 2


def _emb_body(idx_hbm, tbl_hbm, out_hbm, idx_v, buf0, buf1,
              gsem0, gsem1, osem0, osem1):
    nchunk = idx_hbm.shape[1] // CHUNK
    wid = lax.axis_index("s") * NC + lax.axis_index("c")
    pltpu.sync_copy(idx_hbm.at[wid], idx_v)

    bufs = (buf0, buf1)
    gsems = (gsem0, gsem1)
    osems = (osem0, osem1)

    def start_gather(cj, b):
        pltpu.async_copy(tbl_hbm.at[idx_v.at[pl.ds(cj * CHUNK, CHUNK)]],
                         bufs[b], gsems[b])

    def wait_gather(b):
        pltpu.make_async_copy(tbl_hbm.at[idx_v.at[pl.ds(0, CHUNK)]],
                              bufs[b], gsems[b]).wait()

    def start_store(cj, b):
        pltpu.async_copy(bufs[b], out_hbm.at[wid, cj], osems[b])

    def wait_store(b):
        pltpu.make_async_copy(bufs[b], out_hbm.at[wid, 0], osems[b]).wait()

    for b in range(NBUF):
        start_gather(b, b)

    @pl.loop(0, nchunk - NBUF, step=NBUF)
    def _(j):
        for b in range(NBUF):
            wait_gather(b)
            start_store(j + b, b)
        for b in range(NBUF):
            wait_store(b)
            start_gather(j + NBUF + b, b)

    for b in range(NBUF):
        wait_gather(b)
        start_store(nchunk - NBUF + b, b)
    for b in range(NBUF):
        wait_store(b)


def kernel(input_ids, weight):
    bsz, seq = input_ids.shape
    total = bsz * seq
    assert total % (NW * CHUNK) == 0
    per_w = total // NW
    nchunk = per_w // CHUNK
    idx = input_ids.reshape(NW, per_w).astype(jnp.int32)

    mesh = plsc.VectorSubcoreMesh(core_axis_name="c", subcore_axis_name="s")
    f = pl.kernel(
        _emb_body,
        out_type=jax.ShapeDtypeStruct((NW, nchunk, CHUNK, D), jnp.float32),
        mesh=mesh,
        scratch_types=[
            pltpu.VMEM((per_w,), jnp.int32),
            pltpu.VMEM((CHUNK, D), jnp.float32),
            pltpu.VMEM((CHUNK, D), jnp.float32),
            pltpu.SemaphoreType.DMA,
            pltpu.SemaphoreType.DMA,
            pltpu.SemaphoreType.DMA,
            pltpu.SemaphoreType.DMA,
        ],
        compiler_params=pltpu.CompilerParams(use_tc_tiling_on_sc=False),
    )
    out = f(idx, weight)
    return out.reshape(bsz, seq, D)


# speedup vs baseline: 1.0230x; 1.0138x over previous
import jax
import jax.numpy as jnp
from jax import lax
from jax.experimental import pallas as pl
from jax.experimental.pallas import tpu as pltpu
from jax.experimental.pallas import tpu_sc as plsc

V = 1000000
D = 64
S = 50
B = 4096
NC = 2
NS = 16
NW = NC * NS
TB = B // NW
SC_ = 5
NCHUNK = S // SC_
NBUF = 2


def _emb_body(idx_hbm, tbl_hbm, out_hbm, idxv, valv0, valv1,
              ssem, gsem0, gsem1, osem0, osem1):
    c = lax.axis_index("c")
    t = lax.axis_index("s")
    w = t * NC + c

    pltpu.async_copy(idx_hbm.at[:, pl.ds(w * TB, TB)], idxv, ssem).wait()

    valvs = (valv0, valv1)
    gsems = (gsem0, gsem1)
    osems = (osem0, osem1)

    def start_gather(j, b):
        for k in range(SC_):
            pltpu.async_copy(tbl_hbm.at[idxv.at[j * SC_ + k]],
                             valvs[b].at[k], gsems[b])

    def drain_gather(b):
        pltpu.make_async_copy(out_hbm.at[pl.ds(0, SC_), pl.ds(0, TB), :],
                              valvs[b], gsems[b]).wait()

    def start_store(j, b):
        pltpu.async_copy(valvs[b],
                         out_hbm.at[pl.ds(j * SC_, SC_), pl.ds(w * TB, TB), :],
                         osems[b])

    def wait_store(b):
        pltpu.make_async_copy(valvs[b],
                              out_hbm.at[pl.ds(0, SC_), pl.ds(0, TB), :],
                              osems[b]).wait()

    for b in range(NBUF):
        start_gather(b, b)

    @pl.loop(0, NCHUNK - NBUF, step=NBUF)
    def _(j):
        for b in range(NBUF):
            drain_gather(b)
            start_store(j + b, b)
        for b in range(NBUF):
            wait_store(b)
            start_gather(j + NBUF + b, b)

    for b in range(NBUF):
        drain_gather(b)
        start_store(NCHUNK - NBUF + b, b)
    for b in range(NBUF):
        wait_store(b)


def kernel(input_ids, weight):
    idx2 = input_ids.T.astype(jnp.int32)

    mesh = plsc.VectorSubcoreMesh(core_axis_name="c", subcore_axis_name="s")
    f = pl.kernel(
        _emb_body,
        out_type=jax.ShapeDtypeStruct((S, B, D), jnp.float32),
        mesh=mesh,
        scratch_types=[
            pltpu.VMEM((S, TB), jnp.int32),
            pltpu.VMEM((SC_, TB, D), jnp.float32),
            pltpu.VMEM((SC_, TB, D), jnp.float32),
            pltpu.SemaphoreType.DMA,
            pltpu.SemaphoreType.DMA,
            pltpu.SemaphoreType.DMA,
            pltpu.SemaphoreType.DMA,
            pltpu.SemaphoreType.DMA,
        ],
        compiler_params=pltpu.CompilerParams(use_tc_tiling_on_sc=False),
    )
    out = f(idx2, weight)
    return out.transpose(1, 0, 2)


# speedup vs baseline: 1.0304x; 1.0072x over previous
import jax
import jax.numpy as jnp
from jax import lax
from jax.experimental import pallas as pl
from jax.experimental.pallas import tpu as pltpu
from jax.experimental.pallas import tpu_sc as plsc

V = 1000000
D = 64
S = 50
B = 4096
NC = 2
NS = 16
NW = NC * NS
TB = B // NW
NBUF = 2
L = 16


def _emb_body(idx_hbm, tbl_hbm, out_hbm, idxv, hiv,
              valv0, valv1, outv0, outv1,
              ssem, gsem0, gsem1, osem0, osem1):
    c = lax.axis_index("c")
    t = lax.axis_index("s")
    w = t * NC + c

    pltpu.async_copy(idx_hbm.at[:, pl.ds(w * TB, TB)], idxv, ssem).wait()

    @pl.loop(0, S)
    def _(j):
        @pl.loop(0, TB // L)
        def _(g):
            hiv[j, pl.ds(g * L, L)] = lax.shift_right_logical(
                idxv[j, pl.ds(g * L, L)], 1)

    valvs = (valv0, valv1)
    outvs = (outv0, outv1)
    gsems = (gsem0, gsem1)
    osems = (osem0, osem1)

    def start_gather(j, b):
        pltpu.async_copy(tbl_hbm.at[hiv.at[j]], valvs[b], gsems[b])

    def drain_gather(b):
        pltpu.make_async_copy(tbl_hbm.at[hiv.at[0]], valvs[b], gsems[b]).wait()

    def select(j, b):
        valv, outv = valvs[b], outvs[b]

        @pl.loop(0, TB // L)
        def _(g):
            offv = (idxv[j, pl.ds(g * L, L)] & 1) * D
            for kk in range(L):
                off = offv[kk]
                k = g * L + kk
                for cg in range(D // L):
                    outv[k, pl.ds(cg * L, L)] = valv[k, pl.ds(off + cg * L, L)]

    def start_store(j, b):
        pltpu.async_copy(outvs[b],
                         out_hbm.at[j, pl.ds(w * TB, TB), :], osems[b])

    def wait_store(b):
        pltpu.make_async_copy(outvs[b],
                              out_hbm.at[0, pl.ds(0, TB), :], osems[b]).wait()

    for b in range(NBUF):
        start_gather(b, b)

    @pl.loop(0, S - NBUF, step=NBUF)
    def _(j):
        for b in range(NBUF):
            drain_gather(b)
            select(j + b, b)
            start_store(j + b, b)
            start_gather(j + NBUF + b, b)
        for b in range(NBUF):
            wait_store(b)

    for b in range(NBUF):
        drain_gather(b)
        select(S - NBUF + b, b)
        start_store(S - NBUF + b, b)
    for b in range(NBUF):
        wait_store(b)


def kernel(input_ids, weight):
    idx2 = input_ids.T.astype(jnp.int32)
    tbl2 = weight.reshape(V // 2, 2 * D)

    mesh = plsc.VectorSubcoreMesh(core_axis_name="c", subcore_axis_name="s")
    f = pl.kernel(
        _emb_body,
        out_type=jax.ShapeDtypeStruct((S, B, D), jnp.float32),
        mesh=mesh,
        scratch_types=[
            pltpu.VMEM((S, TB), jnp.int32),
            pltpu.VMEM((S, TB), jnp.int32),
            pltpu.VMEM((TB, 2 * D), jnp.float32),
            pltpu.VMEM((TB, 2 * D), jnp.float32),
            pltpu.VMEM((TB, D), jnp.float32),
            pltpu.VMEM((TB, D), jnp.float32),
            pltpu.SemaphoreType.DMA,
            pltpu.SemaphoreType.DMA,
            pltpu.SemaphoreType.DMA,
            pltpu.SemaphoreType.DMA,
            pltpu.SemaphoreType.DMA,
        ],
        compiler_params=pltpu.CompilerParams(use_tc_tiling_on_sc=True),
    )
    out = f(idx2, tbl2)
    return out.transpose(1, 0, 2)


# speedup vs baseline: 1.0348x; 1.0043x over previous
import jax
import jax.numpy as jnp
from jax import lax
from jax.experimental import pallas as pl
from jax.experimental.pallas import tpu as pltpu
from jax.experimental.pallas import tpu_sc as plsc

V = 1000000
D = 64
S = 50
B = 4096
NC = 2
NS = 16
NW = NC * NS
TB = B // NW
NBUF = 3
L = 16


def _emb_body(idx_hbm, tbl_hbm, out_hbm, idxv, hiv,
              valv0, valv1, valv2, outv0, outv1, outv2,
              ssem, gsem0, gsem1, gsem2, osem0, osem1, osem2):
    c = lax.axis_index("c")
    t = lax.axis_index("s")
    w = t * NC + c

    pltpu.async_copy(idx_hbm.at[:, pl.ds(w * TB, TB)], idxv, ssem).wait()

    @pl.loop(0, S)
    def _(j):
        @pl.loop(0, TB // L)
        def _(g):
            hiv[j, pl.ds(g * L, L)] = lax.shift_right_logical(
                idxv[j, pl.ds(g * L, L)], 1)

    valvs = (valv0, valv1, valv2)
    outvs = (outv0, outv1, outv2)
    gsems = (gsem0, gsem1, gsem2)
    osems = (osem0, osem1, osem2)

    def start_gather(j, b):
        pltpu.async_copy(tbl_hbm.at[hiv.at[j]], valvs[b], gsems[b])

    def drain_gather(b):
        pltpu.make_async_copy(tbl_hbm.at[hiv.at[0]], valvs[b], gsems[b]).wait()

    def select(j, b):
        valv, outv = valvs[b], outvs[b]

        @pl.loop(0, TB // L)
        def _(g):
            offv = (idxv[j, pl.ds(g * L, L)] & 1) * D
            for kk in range(L):
                off = offv[kk]
                k = g * L + kk
                for cg in range(D // L):
                    outv[k, pl.ds(cg * L, L)] = valv[k, pl.ds(off + cg * L, L)]

    def start_store(j, b):
        pltpu.async_copy(outvs[b],
                         out_hbm.at[j, pl.ds(w * TB, TB), :], osems[b])

    def wait_store(b):
        pltpu.make_async_copy(outvs[b],
                              out_hbm.at[0, pl.ds(0, TB), :], osems[b]).wait()

    for b in range(NBUF):
        start_gather(b, b)

    REM = S % NBUF
    STEADY = S - REM

    @pl.loop(0, STEADY, step=NBUF)
    def _(j):
        for b in range(NBUF):
            drain_gather(b)
            select(j + b, b)
            start_store(j + b, b)
            nj = j + NBUF + b

            @pl.when(nj < S)
            def _():
                start_gather(nj, b)

        for b in range(NBUF):
            wait_store(b)

    for jj in range(STEADY, S):
        b = jj % NBUF
        drain_gather(b)
        select(jj, b)
        start_store(jj, b)
    for jj in range(STEADY, S):
        wait_store(jj % NBUF)


def kernel(input_ids, weight):
    idx2 = input_ids.T.astype(jnp.int32)
    tbl2 = weight.reshape(V // 2, 2 * D)

    mesh = plsc.VectorSubcoreMesh(core_axis_name="c", subcore_axis_name="s")
    f = pl.kernel(
        _emb_body,
        out_type=jax.ShapeDtypeStruct((S, B, D), jnp.float32),
        mesh=mesh,
        scratch_types=[
            pltpu.VMEM((S, TB), jnp.int32),
            pltpu.VMEM((S, TB), jnp.int32),
            pltpu.VMEM((TB, 2 * D), jnp.float32),
            pltpu.VMEM((TB, 2 * D), jnp.float32),
            pltpu.VMEM((TB, 2 * D), jnp.float32),
            pltpu.VMEM((TB, D), jnp.float32),
            pltpu.VMEM((TB, D), jnp.float32),
            pltpu.VMEM((TB, D), jnp.float32),
            pltpu.SemaphoreType.DMA,
            pltpu.SemaphoreType.DMA,
            pltpu.SemaphoreType.DMA,
            pltpu.SemaphoreType.DMA,
            pltpu.SemaphoreType.DMA,
            pltpu.SemaphoreType.DMA,
            pltpu.SemaphoreType.DMA,
        ],
        compiler_params=pltpu.CompilerParams(use_tc_tiling_on_sc=True),
    )
    out = f(idx2, tbl2)
    return out.transpose(1, 0, 2)
